# baseline (speedup 1.0000x reference)
import jax
import jax.numpy as jnp
from jax.experimental import pallas as pl
from jax.experimental.pallas import tpu as pltpu

_B, _N1, _N2 = 8, 1024, 4096
_C, _CP = 256, 128
_H1, _H2 = 256, 256
_TN2 = 1024
_TB = 2048
_M = _B * _N2
_NT = _N2 // _TN2
_SK = _B * _NT
_NB = _M // _TB
_EPS_BN = 1e-5


def _dot(a, b):
    return jax.lax.dot_general(
        a, b, (((1,), (0,)), ((), ())),
        preferred_element_type=jnp.float32)


def _affine(st_ref, row, g_ref, be_ref):
    mean = st_ref[row:row + 1, :] * (1.0 / _M)
    ex2 = st_ref[row + 1:row + 2, :] * (1.0 / _M)
    var = ex2 - mean * mean
    a = g_ref[...] * jax.lax.rsqrt(var + _EPS_BN)
    c = be_ref[...] - a * mean
    return a, c


def _body(x_ref, w1a_ref, pp_ref, q_ref, px_ref, w1b_ref,
          g1_ref, be1_ref, w2_ref, g2_ref, be2_ref,
          o_ref, st_ref, xw_ref, ybuf_ref):
    s = pl.program_id(0)

    @pl.when(s == 0)
    def _():
        st_ref[...] = jnp.zeros_like(st_ref)

    @pl.when(jnp.logical_and(s < _SK, s % _NT == 0))
    def _():
        xw_ref[...] = _dot(x_ref[0].astype(jnp.bfloat16),
                           w1a_ref[...]).astype(jnp.bfloat16)

    @pl.when(s < _SK)
    def _():
        pp = pp_ref[0]
        q = q_ref[0]
        d2 = None
        for dim in range(3):
            diff = pp[:, dim:dim + 1] - q[dim:dim + 1, :]
            term = diff * diff
            d2 = term if d2 is None else d2 + term

        big = jnp.float32(3.0e38)
        m1 = jnp.min(d2, axis=1, keepdims=True)
        mask1 = d2 == m1
        c2 = jnp.where(mask1, big, d2)
        m2 = jnp.min(c2, axis=1, keepdims=True)
        mask2 = c2 == m2
        c3 = jnp.where(mask2, big, c2)
        m3 = jnp.min(c3, axis=1, keepdims=True)
        mask3 = c3 == m3

        r1 = jax.lax.rsqrt(m1 + 1e-16)
        r2 = jax.lax.rsqrt(m2 + 1e-16)
        r3 = jax.lax.rsqrt(m3 + 1e-16)
        inv_norm = 1.0 / ((r1 + r2) + r3)
        w1 = r1 * inv_norm
        w2 = r2 * inv_norm
        w3 = r3 * inv_norm

        zero = jnp.zeros((), jnp.float32)
        s_mat = (jnp.where(mask1, w1, zero) +
                 jnp.where(mask2, w2, zero) +
                 jnp.where(mask3, w3, zero)).astype(jnp.bfloat16)

        y1 = _dot(s_mat, xw_ref[...]) + _dot(
            px_ref[0].astype(jnp.bfloat16), w1b_ref[...])
        ybuf_ref[pl.ds(s * _TN2, _TN2), :] = y1.astype(jnp.bfloat16)

        st_ref[0:1, :] = st_ref[0:1, :] + jnp.sum(y1, axis=0, keepdims=True)
        st_ref[1:2, :] = st_ref[1:2, :] + jnp.sum(y1 * y1, axis=0,
                                                  keepdims=True)

    @pl.when(jnp.logical_and(s >= _SK, s < _SK + _NB))
    def _():
        g = s - _SK
        a1, c1 = _affine(st_ref, 0, g1_ref, be1_ref)
        y1 = ybuf_ref[pl.ds(g * _TB, _TB), :].astype(jnp.float32)
        h = jnp.maximum(y1 * a1 + c1, 0.0)
        y2 = _dot(h.astype(jnp.bfloat16), w2_ref[...])
        ybuf_ref[pl.ds(g * _TB, _TB), :] = y2.astype(jnp.bfloat16)
        st_ref[2:3, :] = st_ref[2:3, :] + jnp.sum(y2, axis=0, keepdims=True)
        st_ref[3:4, :] = st_ref[3:4, :] + jnp.sum(y2 * y2, axis=0,
                                                  keepdims=True)

    @pl.when(s >= _SK + _NB)
    def _():
        g = s - (_SK + _NB)
        a2, c2 = _affine(st_ref, 2, g2_ref, be2_ref)
        y2 = ybuf_ref[pl.ds(g * _TB, _TB), :].astype(jnp.float32)
        o_ref[...] = jnp.maximum(y2 * a2 + c2, 0.0)


def kernel(x, pos, prev_x, prev_pos, W1, b1, g1, be1, W2, b2, g2, be2):
    f32 = jnp.float32
    bf16 = jnp.bfloat16
    pos_t = jnp.transpose(pos, (0, 2, 1))
    w1a = W1[:_C].astype(bf16)
    w1b = W1[_C:].astype(bf16)
    g1r = g1.reshape(1, _H1)
    be1r = be1.reshape(1, _H1)
    g2r = g2.reshape(1, _H2)
    be2r = be2.reshape(1, _H2)

    def _bidx(s):
        return jnp.minimum(s // _NT, _B - 1)

    def _tidx(s):
        return jnp.where(s < _SK, s % _NT, _NT - 1)

    nsteps = _SK + 2 * _NB
    y, _st = pl.pallas_call(
        _body,
        grid=(nsteps,),
        in_specs=[
            pl.BlockSpec((1, _N1, _C), lambda s: (_bidx(s), 0, 0)),
            pl.BlockSpec((_C, _H1), lambda s: (0, 0)),
            pl.BlockSpec((1, _TN2, 3), lambda s: (_bidx(s), _tidx(s), 0)),
            pl.BlockSpec((1, 3, _N1), lambda s: (_bidx(s), 0, 0)),
            pl.BlockSpec((1, _TN2, _CP), lambda s: (_bidx(s), _tidx(s), 0)),
            pl.BlockSpec((_CP, _H1), lambda s: (0, 0)),
            pl.BlockSpec((1, _H1), lambda s: (0, 0)),
            pl.BlockSpec((1, _H1), lambda s: (0, 0)),
            pl.BlockSpec((_H1, _H2), lambda s: (0, 0)),
            pl.BlockSpec((1, _H2), lambda s: (0, 0)),
            pl.BlockSpec((1, _H2), lambda s: (0, 0)),
        ],
        out_specs=[
            pl.BlockSpec((_TB, _H2),
                         lambda s: (jnp.where(s >= _SK + _NB,
                                              s - (_SK + _NB), 0), 0)),
            pl.BlockSpec((8, _H2), lambda s: (0, 0)),
        ],
        out_shape=[
            jax.ShapeDtypeStruct((_M, _H2), f32),
            jax.ShapeDtypeStruct((8, _H2), f32),
        ],
        scratch_shapes=[
            pltpu.VMEM((_N1, _H1), bf16),
            pltpu.VMEM((_M, _H1), bf16),
        ],
    )(x, w1a, prev_pos, pos_t, prev_x, w1b,
      g1r, be1r, W2.astype(bf16), g2r, be2r)

    return (y.reshape(_B, _N2, _H2), prev_pos)


# speedup vs baseline: 1.0413x; 1.0413x over previous
import jax
import jax.numpy as jnp
from jax.experimental import pallas as pl
from jax.experimental.pallas import tpu as pltpu

_B, _N1, _N2 = 8, 1024, 4096
_C, _CP = 256, 128
_H1, _H2 = 256, 256
_TN2 = 1024
_TB = 2048
_M = _B * _N2
_NT = _N2 // _TN2
_SK = _B * _NT
_NB = _M // _TB
_EPS_BN = 1e-5


def _dot(a, b):
    return jax.lax.dot_general(
        a, b, (((1,), (0,)), ((), ())),
        preferred_element_type=jnp.float32)


def _affine(st_ref, row, g_ref, be_ref):
    mean = st_ref[row:row + 1, :] * (1.0 / _M)
    ex2 = st_ref[row + 1:row + 2, :] * (1.0 / _M)
    var = ex2 - mean * mean
    a = g_ref[...] * jax.lax.rsqrt(var + _EPS_BN)
    c = be_ref[...] - a * mean
    return a, c


def _body(x_ref, w1a_ref, pp_ref, q_ref, px_ref, w1b_ref,
          g1_ref, be1_ref, w2_ref, g2_ref, be2_ref,
          o_ref, st_ref, xw_ref, ybuf_ref):
    s = pl.program_id(0)

    @pl.when(s == 0)
    def _():
        st_ref[...] = jnp.zeros_like(st_ref)

    @pl.when(jnp.logical_and(s < _SK, s % _NT == 0))
    def _():
        xw_ref[...] = _dot(x_ref[0].astype(jnp.bfloat16),
                           w1a_ref[...]).astype(jnp.bfloat16)

    @pl.when(s < _SK)
    def _():
        pp = pp_ref[0]
        q = q_ref[0]
        d2 = None
        for dim in range(3):
            diff = pp[:, dim:dim + 1] - q[dim:dim + 1, :]
            term = diff * diff
            d2 = term if d2 is None else d2 + term

        big = jnp.float32(3.0e38)
        m1 = jnp.min(d2, axis=1, keepdims=True)
        mask1 = d2 == m1
        c2 = jnp.where(mask1, big, d2)
        m2 = jnp.min(c2, axis=1, keepdims=True)
        mask2 = c2 == m2
        c3 = jnp.where(mask2, big, c2)
        m3 = jnp.min(c3, axis=1, keepdims=True)
        mask3 = c3 == m3

        r1 = jax.lax.rsqrt(m1 + 1e-16)
        r2 = jax.lax.rsqrt(m2 + 1e-16)
        r3 = jax.lax.rsqrt(m3 + 1e-16)
        inv_norm = 1.0 / ((r1 + r2) + r3)
        w1 = r1 * inv_norm
        w2 = r2 * inv_norm
        w3 = r3 * inv_norm

        zero = jnp.zeros((), jnp.float32)
        s_mat = jnp.where(
            mask1, w1,
            jnp.where(mask2, w2, jnp.where(mask3, w3, zero))
        ).astype(jnp.bfloat16)

        y1 = _dot(s_mat, xw_ref[...]) + _dot(
            px_ref[0].astype(jnp.bfloat16), w1b_ref[...])
        ybuf_ref[pl.ds(s * _TN2, _TN2), :] = y1.astype(jnp.bfloat16)

        st_ref[0:1, :] = st_ref[0:1, :] + jnp.sum(y1, axis=0, keepdims=True)
        st_ref[1:2, :] = st_ref[1:2, :] + jnp.sum(y1 * y1, axis=0,
                                                  keepdims=True)

    @pl.when(jnp.logical_and(s >= _SK, s < _SK + _NB))
    def _():
        g = s - _SK
        a1, c1 = _affine(st_ref, 0, g1_ref, be1_ref)
        y1 = ybuf_ref[pl.ds(g * _TB, _TB), :].astype(jnp.float32)
        h = jnp.maximum(y1 * a1 + c1, 0.0)
        y2 = _dot(h.astype(jnp.bfloat16), w2_ref[...])
        ybuf_ref[pl.ds(g * _TB, _TB), :] = y2.astype(jnp.bfloat16)
        st_ref[2:3, :] = st_ref[2:3, :] + jnp.sum(y2, axis=0, keepdims=True)
        st_ref[3:4, :] = st_ref[3:4, :] + jnp.sum(y2 * y2, axis=0,
                                                  keepdims=True)

    @pl.when(s >= _SK + _NB)
    def _():
        g = s - (_SK + _NB)
        a2, c2 = _affine(st_ref, 2, g2_ref, be2_ref)
        y2 = ybuf_ref[pl.ds(g * _TB, _TB), :].astype(jnp.float32)
        o_ref[...] = jnp.maximum(y2 * a2 + c2, 0.0)


def kernel(x, pos, prev_x, prev_pos, W1, b1, g1, be1, W2, b2, g2, be2):
    f32 = jnp.float32
    bf16 = jnp.bfloat16
    pos_t = jnp.transpose(pos, (0, 2, 1))
    w1a = W1[:_C].astype(bf16)
    w1b = W1[_C:].astype(bf16)
    g1r = g1.reshape(1, _H1)
    be1r = be1.reshape(1, _H1)
    g2r = g2.reshape(1, _H2)
    be2r = be2.reshape(1, _H2)

    def _bidx(s):
        return jnp.minimum(s // _NT, _B - 1)

    def _tidx(s):
        return jnp.where(s < _SK, s % _NT, _NT - 1)

    nsteps = _SK + 2 * _NB
    y, _st = pl.pallas_call(
        _body,
        grid=(nsteps,),
        in_specs=[
            pl.BlockSpec((1, _N1, _C), lambda s: (_bidx(s), 0, 0)),
            pl.BlockSpec((_C, _H1), lambda s: (0, 0)),
            pl.BlockSpec((1, _TN2, 3), lambda s: (_bidx(s), _tidx(s), 0)),
            pl.BlockSpec((1, 3, _N1), lambda s: (_bidx(s), 0, 0)),
            pl.BlockSpec((1, _TN2, _CP), lambda s: (_bidx(s), _tidx(s), 0)),
            pl.BlockSpec((_CP, _H1), lambda s: (0, 0)),
            pl.BlockSpec((1, _H1), lambda s: (0, 0)),
            pl.BlockSpec((1, _H1), lambda s: (0, 0)),
            pl.BlockSpec((_H1, _H2), lambda s: (0, 0)),
            pl.BlockSpec((1, _H2), lambda s: (0, 0)),
            pl.BlockSpec((1, _H2), lambda s: (0, 0)),
        ],
        out_specs=[
            pl.BlockSpec((_TB, _H2),
                         lambda s: (jnp.where(s >= _SK + _NB,
                                              s - (_SK + _NB), 0), 0)),
            pl.BlockSpec((8, _H2), lambda s: (0, 0)),
        ],
        out_shape=[
            jax.ShapeDtypeStruct((_M, _H2), f32),
            jax.ShapeDtypeStruct((8, _H2), f32),
        ],
        scratch_shapes=[
            pltpu.VMEM((_N1, _H1), bf16),
            pltpu.VMEM((_M, _H1), bf16),
        ],
    )(x, w1a, prev_pos, pos_t, prev_x, w1b,
      g1r, be1r, W2.astype(bf16), g2r, be2r)

    return (y.reshape(_B, _N2, _H2), prev_pos)


# speedup vs baseline: 1.0735x; 1.0309x over previous
import jax
import jax.numpy as jnp
from jax.experimental import pallas as pl
from jax.experimental.pallas import tpu as pltpu

_B, _N1, _N2 = 8, 1024, 4096
_C, _CP = 256, 128
_H1, _H2 = 256, 256
_TN2 = 1024
_TB = 4096
_M = _B * _N2
_NT = _N2 // _TN2
_SK = _B * _NT
_NB = _M // _TB
_EPS_BN = 1e-5


def _dot(a, b):
    return jax.lax.dot_general(
        a, b, (((1,), (0,)), ((), ())),
        preferred_element_type=jnp.float32)


def _affine(st_ref, row, g_ref, be_ref):
    mean = st_ref[row:row + 1, :] * (1.0 / _M)
    ex2 = st_ref[row + 1:row + 2, :] * (1.0 / _M)
    var = ex2 - mean * mean
    a = g_ref[...] * jax.lax.rsqrt(var + _EPS_BN)
    c = be_ref[...] - a * mean
    return a, c


def _body(x_ref, w1a_ref, pp_ref, q_ref, px_ref, w1b_ref,
          g1_ref, be1_ref, w2_ref, g2_ref, be2_ref,
          o_ref, st_ref, xw_ref, ybuf_ref):
    s = pl.program_id(0)

    @pl.when(s == 0)
    def _():
        st_ref[...] = jnp.zeros_like(st_ref)

    @pl.when(jnp.logical_and(s < _SK, s % _NT == 0))
    def _():
        xw_ref[...] = _dot(x_ref[0].astype(jnp.bfloat16),
                           w1a_ref[...]).astype(jnp.bfloat16)

    @pl.when(s < _SK)
    def _():
        pp = pp_ref[0]
        q = q_ref[0]
        d2 = None
        for dim in range(3):
            diff = pp[:, dim:dim + 1] - q[dim:dim + 1, :]
            term = diff * diff
            d2 = term if d2 is None else d2 + term

        big = jnp.float32(3.0e38)
        m1 = jnp.min(d2, axis=1, keepdims=True)
        mask1 = d2 == m1
        c2 = jnp.where(mask1, big, d2)
        m2 = jnp.min(c2, axis=1, keepdims=True)
        mask2 = c2 == m2
        c3 = jnp.where(mask2, big, c2)
        m3 = jnp.min(c3, axis=1, keepdims=True)
        mask3 = c3 == m3

        r1 = jax.lax.rsqrt(m1 + 1e-16)
        r2 = jax.lax.rsqrt(m2 + 1e-16)
        r3 = jax.lax.rsqrt(m3 + 1e-16)
        inv_norm = 1.0 / ((r1 + r2) + r3)
        w1 = r1 * inv_norm
        w2 = r2 * inv_norm
        w3 = r3 * inv_norm

        zero = jnp.zeros((), jnp.float32)
        s_mat = jnp.where(
            mask1, w1,
            jnp.where(mask2, w2, jnp.where(mask3, w3, zero))
        ).astype(jnp.bfloat16)

        y1 = _dot(s_mat, xw_ref[...]) + _dot(
            px_ref[0].astype(jnp.bfloat16), w1b_ref[...])
        ybuf_ref[pl.ds(s * _TN2, _TN2), :] = y1.astype(jnp.bfloat16)

        st_ref[0:1, :] = st_ref[0:1, :] + jnp.sum(y1, axis=0, keepdims=True)
        st_ref[1:2, :] = st_ref[1:2, :] + jnp.sum(y1 * y1, axis=0,
                                                  keepdims=True)

    @pl.when(jnp.logical_and(s >= _SK, s < _SK + _NB))
    def _():
        g = s - _SK
        a1, c1 = _affine(st_ref, 0, g1_ref, be1_ref)
        y1 = ybuf_ref[pl.ds(g * _TB, _TB), :].astype(jnp.float32)
        h = jnp.maximum(y1 * a1 + c1, 0.0)
        y2 = _dot(h.astype(jnp.bfloat16), w2_ref[...])
        ybuf_ref[pl.ds(g * _TB, _TB), :] = y2.astype(jnp.bfloat16)
        st_ref[2:3, :] = st_ref[2:3, :] + jnp.sum(y2, axis=0, keepdims=True)
        st_ref[3:4, :] = st_ref[3:4, :] + jnp.sum(y2 * y2, axis=0,
                                                  keepdims=True)

    @pl.when(s >= _SK + _NB)
    def _():
        g = s - (_SK + _NB)
        a2, c2 = _affine(st_ref, 2, g2_ref, be2_ref)
        y2 = ybuf_ref[pl.ds(g * _TB, _TB), :].astype(jnp.float32)
        o_ref[...] = jnp.maximum(y2 * a2 + c2, 0.0)


def kernel(x, pos, prev_x, prev_pos, W1, b1, g1, be1, W2, b2, g2, be2):
    f32 = jnp.float32
    bf16 = jnp.bfloat16
    pos_t = jnp.transpose(pos, (0, 2, 1))
    w1a = W1[:_C].astype(bf16)
    w1b = W1[_C:].astype(bf16)
    g1r = g1.reshape(1, _H1)
    be1r = be1.reshape(1, _H1)
    g2r = g2.reshape(1, _H2)
    be2r = be2.reshape(1, _H2)

    def _bidx(s):
        return jnp.minimum(s // _NT, _B - 1)

    def _tidx(s):
        return jnp.where(s < _SK, s % _NT, _NT - 1)

    nsteps = _SK + 2 * _NB
    y, _st = pl.pallas_call(
        _body,
        grid=(nsteps,),
        in_specs=[
            pl.BlockSpec((1, _N1, _C), lambda s: (_bidx(s), 0, 0)),
            pl.BlockSpec((_C, _H1), lambda s: (0, 0)),
            pl.BlockSpec((1, _TN2, 3), lambda s: (_bidx(s), _tidx(s), 0)),
            pl.BlockSpec((1, 3, _N1), lambda s: (_bidx(s), 0, 0)),
            pl.BlockSpec((1, _TN2, _CP), lambda s: (_bidx(s), _tidx(s), 0)),
            pl.BlockSpec((_CP, _H1), lambda s: (0, 0)),
            pl.BlockSpec((1, _H1), lambda s: (0, 0)),
            pl.BlockSpec((1, _H1), lambda s: (0, 0)),
            pl.BlockSpec((_H1, _H2), lambda s: (0, 0)),
            pl.BlockSpec((1, _H2), lambda s: (0, 0)),
            pl.BlockSpec((1, _H2), lambda s: (0, 0)),
        ],
        out_specs=[
            pl.BlockSpec((_TB, _H2),
                         lambda s: (jnp.where(s >= _SK + _NB,
                                              s - (_SK + _NB), 0), 0)),
            pl.BlockSpec((8, _H2), lambda s: (0, 0)),
        ],
        out_shape=[
            jax.ShapeDtypeStruct((_M, _H2), f32),
            jax.ShapeDtypeStruct((8, _H2), f32),
        ],
        scratch_shapes=[
            pltpu.VMEM((_N1, _H1), bf16),
            pltpu.VMEM((_M, _H1), bf16),
        ],
    )(x, w1a, prev_pos, pos_t, prev_x, w1b,
      g1r, be1r, W2.astype(bf16), g2r, be2r)

    return (y.reshape(_B, _N2, _H2), prev_pos)
